# Initial kernel scaffold; baseline (speedup 1.0000x reference)
import functools

import jax
import jax.numpy as jnp
from jax import lax
from jax.experimental import pallas as pl
from jax.experimental.pallas import tpu as pltpu
from jax.experimental.pallas import tpu_sc as plsc

N = 10000
E = 320000
D = 128
H = 64
G = 128

NW = 32
CHUNK = 128
CPW = 79
EPAD = NW * CPW * CHUNK
NPAD = 10112
ACC = 10240
TPT = ACC // 16

_F32 = jnp.float32


def _mm(a, b):
    return jax.lax.dot_general(a, b, (((1,), (0,)), ((), ())),
                               preferred_element_type=_F32)


def _tc1_body(x_ref, wl_ref, wr_ref, p_ref, xr_ref):
    xb = x_ref[...]
    p_ref[...] = _mm(xb, wl_ref[...])
    xr_ref[...] = _mm(xb, wr_ref[...])


_tc1 = pl.pallas_call(
    _tc1_body,
    grid=(8,),
    in_specs=[
        pl.BlockSpec((1264, D), lambda i: (i, 0)),
        pl.BlockSpec((D, H), lambda i: (0, 0)),
        pl.BlockSpec((D, H), lambda i: (0, 0)),
    ],
    out_specs=[
        pl.BlockSpec((1264, H), lambda i: (i, 0)),
        pl.BlockSpec((1264, H), lambda i: (i, 0)),
    ],
    out_shape=[
        jax.ShapeDtypeStruct((NPAD, H), _F32),
        jax.ShapeDtypeStruct((NPAD, H), _F32),
    ],
)


def _sc_body(with_counts, *refs):
    if with_counts:
        (p_hbm, srcm, dstm, out_s, out_c,
         idxs, idxd, rows, ones_v, zbuf, zbufc, acc, cacc, sem) = refs
    else:
        (p_hbm, srcm, dstm, out_s,
         idxs, idxd, rows, zbuf, acc, sem) = refs

    cid = lax.axis_index("c")
    sid = lax.axis_index("s")
    wid = sid * 2 + cid
    base = sid * TPT

    zero16 = jnp.zeros((16,), _F32)

    def zfill(i, c):
        for j in range(4):
            zbuf[i, pl.ds(16 * j, 16)] = zero16
        if with_counts:
            zbufc[i, pl.ds(0, 16)] = zero16
            ones_v[i, pl.ds(0, 16)] = jnp.ones((16,), _F32)
            ones_v[i + 64, pl.ds(0, 16)] = jnp.ones((16,), _F32)
        return c

    lax.fori_loop(0, 64, zfill, 0)

    def zcopy(k, c):
        pltpu.sync_copy(zbuf, acc.at[pl.ds(base + k * 64, 64)])
        if with_counts:
            pltpu.sync_copy(zbufc, cacc.at[pl.ds(base + k * 64, 64)])
        return c

    lax.fori_loop(0, TPT // 64, zcopy, 0)
    plsc.subcore_barrier()

    pltpu.sync_copy(srcm.at[pl.ds(wid * CPW, CPW)], idxs)
    pltpu.sync_copy(dstm.at[pl.ds(wid * CPW, CPW)], idxd)

    def edge(j, c):
        pltpu.async_copy(p_hbm.at[idxs.at[j]], rows, sem).wait()
        pltpu.sync_copy(rows, acc.at[idxd.at[j]], add=True)
        if with_counts:
            pltpu.sync_copy(ones_v, cacc.at[idxd.at[j]], add=True)
        return c

    lax.fori_loop(0, CPW, edge, 0)
    plsc.subcore_barrier()

    pltpu.sync_copy(acc.at[pl.ds(base, TPT)], out_s.at[cid, pl.ds(base, TPT)])
    if with_counts:
        pltpu.sync_copy(cacc.at[pl.ds(base, TPT)],
                        out_c.at[cid, pl.ds(base, TPT)])


def _make_sc(with_counts):
    mesh = plsc.VectorSubcoreMesh(core_axis_name="c", subcore_axis_name="s")
    out_type = [jax.ShapeDtypeStruct((2, ACC, H), _F32)]
    scratch = [
        pltpu.VMEM((CPW, CHUNK), jnp.int32),
        pltpu.VMEM((CPW, CHUNK), jnp.int32),
        pltpu.VMEM((CHUNK, H), _F32),
    ]
    if with_counts:
        out_type.append(jax.ShapeDtypeStruct((2, ACC, 16), _F32))
        scratch.append(pltpu.VMEM((CHUNK, 16), _F32))
    scratch.append(pltpu.VMEM((64, H), _F32))
    if with_counts:
        scratch.append(pltpu.VMEM((64, 16), _F32))
    scratch.append(pltpu.VMEM_SHARED((ACC, H), _F32))
    if with_counts:
        scratch.append(pltpu.VMEM_SHARED((ACC, 16), _F32))
    scratch.append(pltpu.SemaphoreType.DMA)
    return pl.kernel(
        functools.partial(_sc_body, with_counts),
        out_type=out_type,
        mesh=mesh,
        scratch_types=scratch,
    )


_sc_edge_cnt = _make_sc(True)
_sc_edge = _make_sc(False)


def _mid_body(s_ref, c_ref, xr_ref, b1_ref, wl_ref, wr_ref, p2_ref, h1r_ref):
    s = s_ref[0] + s_ref[1]
    cnt = c_ref[0, :, 0:1] + c_ref[1, :, 0:1]
    h1 = jnp.maximum(s / jnp.maximum(cnt, 1.0) + xr_ref[...] + b1_ref[...],
                     0.0)
    p2_ref[...] = _mm(h1, wl_ref[...])
    h1r_ref[...] = _mm(h1, wr_ref[...])


_tc_mid = pl.pallas_call(
    _mid_body,
    grid=(8,),
    in_specs=[
        pl.BlockSpec((2, 1264, H), lambda i: (0, i, 0)),
        pl.BlockSpec((2, 1264, 16), lambda i: (0, i, 0)),
        pl.BlockSpec((1264, H), lambda i: (i, 0)),
        pl.BlockSpec((1, H), lambda i: (0, 0)),
        pl.BlockSpec((H, H), lambda i: (0, 0)),
        pl.BlockSpec((H, H), lambda i: (0, 0)),
    ],
    out_specs=[
        pl.BlockSpec((1264, H), lambda i: (i, 0)),
        pl.BlockSpec((1264, H), lambda i: (i, 0)),
    ],
    out_shape=[
        jax.ShapeDtypeStruct((NPAD, H), _F32),
        jax.ShapeDtypeStruct((NPAD, H), _F32),
    ],
)


def _post_body(s_ref, c_ref, h1r_ref, b2_ref, bat_ref, wo_ref, bo_ref,
               out_ref, psum, pcnt):
    i = pl.program_id(0)
    s = s_ref[0] + s_ref[1]
    cnt = c_ref[0, :, 0:1] + c_ref[1, :, 0:1]
    h2 = jnp.maximum(s / jnp.maximum(cnt, 1.0) + h1r_ref[...] + b2_ref[...],
                     0.0)
    bcol = bat_ref[...]
    gids = jax.lax.broadcasted_iota(_F32, (1, G), 1)
    onehot = (bcol == gids).astype(_F32)
    ps = jax.lax.dot_general(onehot, h2, (((0,), (0,)), ((), ())),
                             preferred_element_type=_F32)
    ones_col = jnp.ones_like(bcol)
    pc = jax.lax.dot_general(onehot, ones_col, (((0,), (0,)), ((), ())),
                             preferred_element_type=_F32)

    @pl.when(i == 0)
    def _():
        psum[...] = ps
        pcnt[...] = pc

    @pl.when(i > 0)
    def _():
        psum[...] += ps
        pcnt[...] += pc

    @pl.when(i == 9)
    def _():
        pooled = psum[...] / jnp.maximum(pcnt[...], 1.0)
        out_ref[...] = _mm(pooled, wo_ref[...]) + bo_ref[...]


_tc_post = pl.pallas_call(
    _post_body,
    grid=(10,),
    in_specs=[
        pl.BlockSpec((2, 1000, H), lambda i: (0, i, 0)),
        pl.BlockSpec((2, 1000, 16), lambda i: (0, i, 0)),
        pl.BlockSpec((1000, H), lambda i: (i, 0)),
        pl.BlockSpec((1, H), lambda i: (0, 0)),
        pl.BlockSpec((1000, 1), lambda i: (i, 0)),
        pl.BlockSpec((H, 2), lambda i: (0, 0)),
        pl.BlockSpec((1, 2), lambda i: (0, 0)),
    ],
    out_specs=pl.BlockSpec((G, 2), lambda i: (0, 0)),
    out_shape=jax.ShapeDtypeStruct((G, 2), _F32),
    scratch_shapes=[
        pltpu.VMEM((G, H), _F32),
        pltpu.VMEM((G, 1), _F32),
    ],
)


def kernel(x, edge_index, batch, W1_l, b1, W1_r, W2_l, b2, W2_r, W_out, b_out):
    src = edge_index[0]
    dst = edge_index[1]
    srcm = jnp.full((EPAD,), N, jnp.int32).at[:E].set(src).reshape(-1, CHUNK)
    dstm = jnp.full((EPAD,), N, jnp.int32).at[:E].set(dst).reshape(-1, CHUNK)
    x_pad = jnp.zeros((NPAD, D), _F32).at[:N].set(x)
    bat_f = batch.astype(_F32).reshape(N, 1)

    p1, xr = _tc1(x_pad, W1_l, W1_r)
    s1, c1 = _sc_edge_cnt(p1, srcm, dstm)
    p2, h1r = _tc_mid(s1, c1, xr, b1.reshape(1, H), W2_l, W2_r)
    (s2,) = _sc_edge(p2, srcm, dstm)
    return _tc_post(s2, c1, h1r, b2.reshape(1, H), bat_f,
                    W_out, b_out.reshape(1, 2))


# speedup vs baseline: 5.9912x; 5.9912x over previous
import functools

import jax
import jax.numpy as jnp
from jax import lax
from jax.experimental import pallas as pl
from jax.experimental.pallas import tpu as pltpu
from jax.experimental.pallas import tpu_sc as plsc

N = 10000
E = 320000
D = 128
H = 64
G = 128

NW = 32
CHUNK = 128
CPW = 80
EPAD = NW * CPW * CHUNK
NPAD = 10112
ACC = 10240
TPT = ACC // 16

_F32 = jnp.float32


def _mm(a, b):
    return jax.lax.dot_general(a, b, (((1,), (0,)), ((), ())),
                               preferred_element_type=_F32,
                               precision=jax.lax.Precision.HIGHEST)


def _tc1_body(x_ref, wl_ref, wr_ref, p_ref, xr_ref):
    xb = x_ref[...]
    p_ref[...] = _mm(xb, wl_ref[...])
    xr_ref[...] = _mm(xb, wr_ref[...])


_tc1 = pl.pallas_call(
    _tc1_body,
    grid=(8,),
    in_specs=[
        pl.BlockSpec((1264, D), lambda i: (i, 0)),
        pl.BlockSpec((D, H), lambda i: (0, 0)),
        pl.BlockSpec((D, H), lambda i: (0, 0)),
    ],
    out_specs=[
        pl.BlockSpec((1264, H), lambda i: (i, 0)),
        pl.BlockSpec((1264, H), lambda i: (i, 0)),
    ],
    out_shape=[
        jax.ShapeDtypeStruct((NPAD, H), _F32),
        jax.ShapeDtypeStruct((NPAD, H), _F32),
    ],
)


def _sc_body(with_counts, *refs):
    if with_counts:
        (p_hbm, srcm, dstm, out_s, out_c,
         idxs, idxd, rows, ones_v, zbuf, zbufc, acc, cacc, sem) = refs
    else:
        (p_hbm, srcm, dstm, out_s,
         idxs, idxd, rows, zbuf, acc, sem) = refs

    cid = lax.axis_index("c")
    sid = lax.axis_index("s")
    wid = sid * 2 + cid
    base = sid * TPT

    zero16 = jnp.zeros((16,), _F32)

    def zfill(i, c):
        for j in range(4):
            zbuf[i, pl.ds(16 * j, 16)] = zero16
        if with_counts:
            zbufc[i, pl.ds(0, 16)] = zero16
            ones_v[i, pl.ds(0, 16)] = jnp.ones((16,), _F32)
            ones_v[i + 64, pl.ds(0, 16)] = jnp.ones((16,), _F32)
        return c

    lax.fori_loop(0, 64, zfill, 0)

    def zcopy(k, c):
        pltpu.sync_copy(zbuf, acc.at[pl.ds(base + k * 64, 64)])
        if with_counts:
            pltpu.sync_copy(zbufc, cacc.at[pl.ds(base + k * 64, 64)])
        return c

    lax.fori_loop(0, TPT // 64, zcopy, 0)
    plsc.subcore_barrier()

    pltpu.sync_copy(srcm.at[wid], idxs)
    pltpu.sync_copy(dstm.at[wid], idxd)

    def edge(j, c):
        pltpu.async_copy(p_hbm.at[idxs.at[j]], rows, sem).wait()
        pltpu.sync_copy(rows, acc.at[idxd.at[j]], add=True)
        if with_counts:
            pltpu.sync_copy(ones_v, cacc.at[idxd.at[j]], add=True)
        return c

    lax.fori_loop(0, CPW, edge, 0)
    plsc.subcore_barrier()

    pltpu.sync_copy(acc.at[pl.ds(base, TPT)], out_s.at[cid, pl.ds(base, TPT)])
    if with_counts:
        pltpu.sync_copy(cacc.at[pl.ds(base, TPT)],
                        out_c.at[cid, pl.ds(base, TPT)])


def _make_sc(with_counts):
    mesh = plsc.VectorSubcoreMesh(core_axis_name="c", subcore_axis_name="s",
                                  num_cores=2, num_subcores=16)
    out_type = [jax.ShapeDtypeStruct((2, ACC, H), _F32)]
    scratch = [
        pltpu.VMEM((CPW, CHUNK), jnp.int32),
        pltpu.VMEM((CPW, CHUNK), jnp.int32),
        pltpu.VMEM((CHUNK, H), _F32),
    ]
    if with_counts:
        out_type.append(jax.ShapeDtypeStruct((2, ACC, 16), _F32))
        scratch.append(pltpu.VMEM((CHUNK, 16), _F32))
    scratch.append(pltpu.VMEM((64, H), _F32))
    if with_counts:
        scratch.append(pltpu.VMEM((64, 16), _F32))
    scratch.append(pltpu.VMEM_SHARED((ACC, H), _F32))
    if with_counts:
        scratch.append(pltpu.VMEM_SHARED((ACC, 16), _F32))
    scratch.append(pltpu.SemaphoreType.DMA)
    return pl.kernel(
        functools.partial(_sc_body, with_counts),
        out_type=out_type,
        mesh=mesh,
        scratch_types=scratch,
        compiler_params=pltpu.CompilerParams(use_tc_tiling_on_sc=False),
    )


_sc_edge_cnt = _make_sc(True)
_sc_edge = _make_sc(False)


def _mid_body(s_ref, c_ref, xr_ref, b1_ref, wl_ref, wr_ref, p2_ref, h1r_ref):
    s = s_ref[0] + s_ref[1]
    cnt = c_ref[0, :, 0:1] + c_ref[1, :, 0:1]
    h1 = jnp.maximum(s / jnp.maximum(cnt, 1.0) + xr_ref[...] + b1_ref[...],
                     0.0)
    p2_ref[...] = _mm(h1, wl_ref[...])
    h1r_ref[...] = _mm(h1, wr_ref[...])


_tc_mid = pl.pallas_call(
    _mid_body,
    grid=(8,),
    in_specs=[
        pl.BlockSpec((2, 1264, H), lambda i: (0, i, 0)),
        pl.BlockSpec((2, 1264, 16), lambda i: (0, i, 0)),
        pl.BlockSpec((1264, H), lambda i: (i, 0)),
        pl.BlockSpec((1, H), lambda i: (0, 0)),
        pl.BlockSpec((H, H), lambda i: (0, 0)),
        pl.BlockSpec((H, H), lambda i: (0, 0)),
    ],
    out_specs=[
        pl.BlockSpec((1264, H), lambda i: (i, 0)),
        pl.BlockSpec((1264, H), lambda i: (i, 0)),
    ],
    out_shape=[
        jax.ShapeDtypeStruct((NPAD, H), _F32),
        jax.ShapeDtypeStruct((NPAD, H), _F32),
    ],
)


def _post_body(s_ref, c_ref, h1r_ref, b2_ref, bat_ref, wo_ref, bo_ref,
               out_ref, psum, pcnt):
    i = pl.program_id(0)
    s = s_ref[0] + s_ref[1]
    cnt = c_ref[0, :, 0:1] + c_ref[1, :, 0:1]
    h2 = jnp.maximum(s / jnp.maximum(cnt, 1.0) + h1r_ref[...] + b2_ref[...],
                     0.0)
    bcol = bat_ref[...]
    gids = jax.lax.broadcasted_iota(jnp.int32, (1, G), 1).astype(_F32)
    onehot = (bcol == gids).astype(_F32)
    ps = jax.lax.dot_general(onehot, h2, (((0,), (0,)), ((), ())),
                             preferred_element_type=_F32,
                             precision=jax.lax.Precision.HIGHEST)
    ones_col = jnp.ones_like(bcol)
    pc = jax.lax.dot_general(onehot, ones_col, (((0,), (0,)), ((), ())),
                             preferred_element_type=_F32,
                             precision=jax.lax.Precision.HIGHEST)

    @pl.when(i == 0)
    def _():
        psum[...] = ps
        pcnt[...] = pc

    @pl.when(i > 0)
    def _():
        psum[...] += ps
        pcnt[...] += pc

    @pl.when(i == 9)
    def _():
        pooled = psum[...] / jnp.maximum(pcnt[...], 1.0)
        out_ref[...] = _mm(pooled, wo_ref[...]) + bo_ref[...]


_tc_post = pl.pallas_call(
    _post_body,
    grid=(10,),
    in_specs=[
        pl.BlockSpec((2, 1000, H), lambda i: (0, i, 0)),
        pl.BlockSpec((2, 1000, 16), lambda i: (0, i, 0)),
        pl.BlockSpec((1000, H), lambda i: (i, 0)),
        pl.BlockSpec((1, H), lambda i: (0, 0)),
        pl.BlockSpec((1000, 1), lambda i: (i, 0)),
        pl.BlockSpec((H, 2), lambda i: (0, 0)),
        pl.BlockSpec((1, 2), lambda i: (0, 0)),
    ],
    out_specs=pl.BlockSpec((G, 2), lambda i: (0, 0)),
    out_shape=jax.ShapeDtypeStruct((G, 2), _F32),
    scratch_shapes=[
        pltpu.VMEM((G, H), _F32),
        pltpu.VMEM((G, 1), _F32),
    ],
)


def kernel(x, edge_index, batch, W1_l, b1, W1_r, W2_l, b2, W2_r, W_out, b_out):
    src = edge_index[0]
    dst = edge_index[1]
    srcm = (jnp.full((EPAD,), N, jnp.int32).at[:E].set(src)
            .reshape(NW, CPW, CHUNK))
    dstm = (jnp.full((EPAD,), N, jnp.int32).at[:E].set(dst)
            .reshape(NW, CPW, CHUNK))
    x_pad = jnp.zeros((NPAD, D), _F32).at[:N].set(x)
    bat_f = batch.astype(_F32).reshape(N, 1)

    p1, xr = _tc1(x_pad, W1_l, W1_r)
    s1, c1 = _sc_edge_cnt(p1, srcm, dstm)
    p2, h1r = _tc_mid(s1, c1, xr, b1.reshape(1, H), W2_l, W2_r)
    (s2,) = _sc_edge(p2, srcm, dstm)
    return _tc_post(s2, c1, h1r, b2.reshape(1, H), bat_f,
                    W_out, b_out.reshape(1, 2))


# speedup vs baseline: 6.5587x; 1.0947x over previous
import functools

import jax
import jax.numpy as jnp
from jax import lax
from jax.experimental import pallas as pl
from jax.experimental.pallas import tpu as pltpu
from jax.experimental.pallas import tpu_sc as plsc

N = 10000
E = 320000
D = 128
H = 64
G = 128

NW = 32
CHUNK = 128
CPW = 80
EPAD = NW * CPW * CHUNK
NPAD = 10112
ACC = 10240
TPT = ACC // 16

_F32 = jnp.float32


def _mm(a, b):
    return jax.lax.dot_general(a, b, (((1,), (0,)), ((), ())),
                               preferred_element_type=_F32,
                               precision=jax.lax.Precision.HIGHEST)


def _tc1_body(x_ref, wl_ref, wr_ref, p_ref, xr_ref):
    xb = x_ref[...]
    p_ref[...] = _mm(xb, wl_ref[...])
    xr_ref[...] = _mm(xb, wr_ref[...])


_tc1 = pl.pallas_call(
    _tc1_body,
    grid=(8,),
    in_specs=[
        pl.BlockSpec((1264, D), lambda i: (i, 0)),
        pl.BlockSpec((D, H), lambda i: (0, 0)),
        pl.BlockSpec((D, H), lambda i: (0, 0)),
    ],
    out_specs=[
        pl.BlockSpec((1264, H), lambda i: (i, 0)),
        pl.BlockSpec((1264, H), lambda i: (i, 0)),
    ],
    out_shape=[
        jax.ShapeDtypeStruct((NPAD, H), _F32),
        jax.ShapeDtypeStruct((NPAD, H), _F32),
    ],
)


def _sc_body(with_counts, *refs):
    if with_counts:
        (p_hbm, srcm, dstm, out_s, out_c,
         idxs, idxd, rows0, rows1, ones_v, zbuf, zbufc, acc, cacc,
         semg0, semg1, sems0, sems1, semc0, semc1) = refs
    else:
        (p_hbm, srcm, dstm, out_s,
         idxs, idxd, rows0, rows1, zbuf, acc,
         semg0, semg1, sems0, sems1) = refs

    cid = lax.axis_index("c")
    sid = lax.axis_index("s")
    wid = sid * 2 + cid
    base = sid * TPT

    zero16 = jnp.zeros((16,), _F32)

    def zfill(i, c):
        for j in range(4):
            zbuf[i, pl.ds(16 * j, 16)] = zero16
        if with_counts:
            zbufc[i, pl.ds(0, 16)] = zero16
            ones_v[i, pl.ds(0, 16)] = jnp.ones((16,), _F32)
            ones_v[i + 64, pl.ds(0, 16)] = jnp.ones((16,), _F32)
        return c

    lax.fori_loop(0, 64, zfill, 0)

    def zcopy(k, c):
        pltpu.sync_copy(zbuf, acc.at[pl.ds(base + k * 64, 64)])
        if with_counts:
            pltpu.sync_copy(zbufc, cacc.at[pl.ds(base + k * 64, 64)])
        return c

    lax.fori_loop(0, TPT // 64, zcopy, 0)
    plsc.subcore_barrier()

    pltpu.sync_copy(srcm.at[wid], idxs)
    pltpu.sync_copy(dstm.at[wid], idxd)

    def gat(j, rows, sem):
        return pltpu.async_copy(p_hbm.at[idxs.at[j]], rows, sem)

    def gat_wait(j, rows, sem):
        pltpu.make_async_copy(p_hbm.at[idxs.at[j]], rows, sem).wait()

    def sca(j, rows, sem):
        return pltpu.async_copy(rows, acc.at[idxd.at[j]], sem, add=True)

    def sca_wait(j, rows, sem):
        pltpu.make_async_copy(rows, acc.at[idxd.at[j]], sem).wait()

    def cnt(j, sem):
        return pltpu.async_copy(ones_v, cacc.at[idxd.at[j]], sem, add=True)

    def cnt_wait(j, sem):
        pltpu.make_async_copy(ones_v, cacc.at[idxd.at[j]], sem).wait()

    gat(0, rows0, semg0)
    gat(1, rows1, semg1)

    def edge_pair(k, c):
        j = 2 * k
        gat_wait(j, rows0, semg0)
        sca(j, rows0, sems0)
        if with_counts:
            cnt(j, semc0)
        gat_wait(j + 1, rows1, semg1)
        sca(j + 1, rows1, sems1)
        if with_counts:
            cnt(j + 1, semc1)
        sca_wait(j, rows0, sems0)
        if with_counts:
            cnt_wait(j, semc0)
        gat(j + 2, rows0, semg0)
        sca_wait(j + 1, rows1, sems1)
        if with_counts:
            cnt_wait(j + 1, semc1)
        gat(j + 3, rows1, semg1)
        return c

    lax.fori_loop(0, CPW // 2 - 1, edge_pair, 0)

    j = CPW - 2
    gat_wait(j, rows0, semg0)
    sca(j, rows0, sems0)
    if with_counts:
        cnt(j, semc0)
    gat_wait(j + 1, rows1, semg1)
    sca(j + 1, rows1, sems1)
    if with_counts:
        cnt(j + 1, semc1)
    sca_wait(j, rows0, sems0)
    sca_wait(j + 1, rows1, sems1)
    if with_counts:
        cnt_wait(j, semc0)
        cnt_wait(j + 1, semc1)

    plsc.subcore_barrier()

    pltpu.sync_copy(acc.at[pl.ds(base, TPT)], out_s.at[cid, pl.ds(base, TPT)])
    if with_counts:
        pltpu.sync_copy(cacc.at[pl.ds(base, TPT)],
                        out_c.at[cid, pl.ds(base, TPT)])


def _make_sc(with_counts):
    mesh = plsc.VectorSubcoreMesh(core_axis_name="c", subcore_axis_name="s",
                                  num_cores=2, num_subcores=16)
    out_type = [jax.ShapeDtypeStruct((2, ACC, H), _F32)]
    scratch = [
        pltpu.VMEM((CPW, CHUNK), jnp.int32),
        pltpu.VMEM((CPW, CHUNK), jnp.int32),
        pltpu.VMEM((CHUNK, H), _F32),
        pltpu.VMEM((CHUNK, H), _F32),
    ]
    if with_counts:
        out_type.append(jax.ShapeDtypeStruct((2, ACC, 16), _F32))
        scratch.append(pltpu.VMEM((CHUNK, 16), _F32))
    scratch.append(pltpu.VMEM((64, H), _F32))
    if with_counts:
        scratch.append(pltpu.VMEM((64, 16), _F32))
    scratch.append(pltpu.VMEM_SHARED((ACC, H), _F32))
    if with_counts:
        scratch.append(pltpu.VMEM_SHARED((ACC, 16), _F32))
    nsem = 6 if with_counts else 4
    scratch.extend([pltpu.SemaphoreType.DMA] * nsem)
    return pl.kernel(
        functools.partial(_sc_body, with_counts),
        out_type=out_type,
        mesh=mesh,
        scratch_types=scratch,
        compiler_params=pltpu.CompilerParams(use_tc_tiling_on_sc=False),
    )


_sc_edge_cnt = _make_sc(True)
_sc_edge = _make_sc(False)


def _mid_body(s_ref, c_ref, xr_ref, b1_ref, wl_ref, wr_ref, p2_ref, h1r_ref):
    s = s_ref[0] + s_ref[1]
    cnt = c_ref[0, :, 0:1] + c_ref[1, :, 0:1]
    h1 = jnp.maximum(s / jnp.maximum(cnt, 1.0) + xr_ref[...] + b1_ref[...],
                     0.0)
    p2_ref[...] = _mm(h1, wl_ref[...])
    h1r_ref[...] = _mm(h1, wr_ref[...])


_tc_mid = pl.pallas_call(
    _mid_body,
    grid=(8,),
    in_specs=[
        pl.BlockSpec((2, 1264, H), lambda i: (0, i, 0)),
        pl.BlockSpec((2, 1264, 16), lambda i: (0, i, 0)),
        pl.BlockSpec((1264, H), lambda i: (i, 0)),
        pl.BlockSpec((1, H), lambda i: (0, 0)),
        pl.BlockSpec((H, H), lambda i: (0, 0)),
        pl.BlockSpec((H, H), lambda i: (0, 0)),
    ],
    out_specs=[
        pl.BlockSpec((1264, H), lambda i: (i, 0)),
        pl.BlockSpec((1264, H), lambda i: (i, 0)),
    ],
    out_shape=[
        jax.ShapeDtypeStruct((NPAD, H), _F32),
        jax.ShapeDtypeStruct((NPAD, H), _F32),
    ],
)


def _post_body(s_ref, c_ref, h1r_ref, b2_ref, bat_ref, wo_ref, bo_ref,
               out_ref, psum, pcnt):
    i = pl.program_id(0)
    s = s_ref[0] + s_ref[1]
    cnt = c_ref[0, :, 0:1] + c_ref[1, :, 0:1]
    h2 = jnp.maximum(s / jnp.maximum(cnt, 1.0) + h1r_ref[...] + b2_ref[...],
                     0.0)
    bcol = bat_ref[...]
    gids = jax.lax.broadcasted_iota(jnp.int32, (1, G), 1).astype(_F32)
    onehot = (bcol == gids).astype(_F32)
    ps = jax.lax.dot_general(onehot, h2, (((0,), (0,)), ((), ())),
                             preferred_element_type=_F32,
                             precision=jax.lax.Precision.HIGHEST)
    ones_col = jnp.ones_like(bcol)
    pc = jax.lax.dot_general(onehot, ones_col, (((0,), (0,)), ((), ())),
                             preferred_element_type=_F32,
                             precision=jax.lax.Precision.HIGHEST)

    @pl.when(i == 0)
    def _():
        psum[...] = ps
        pcnt[...] = pc

    @pl.when(i > 0)
    def _():
        psum[...] += ps
        pcnt[...] += pc

    @pl.when(i == 9)
    def _():
        pooled = psum[...] / jnp.maximum(pcnt[...], 1.0)
        out_ref[...] = _mm(pooled, wo_ref[...]) + bo_ref[...]


_tc_post = pl.pallas_call(
    _post_body,
    grid=(10,),
    in_specs=[
        pl.BlockSpec((2, 1000, H), lambda i: (0, i, 0)),
        pl.BlockSpec((2, 1000, 16), lambda i: (0, i, 0)),
        pl.BlockSpec((1000, H), lambda i: (i, 0)),
        pl.BlockSpec((1, H), lambda i: (0, 0)),
        pl.BlockSpec((1000, 1), lambda i: (i, 0)),
        pl.BlockSpec((H, 2), lambda i: (0, 0)),
        pl.BlockSpec((1, 2), lambda i: (0, 0)),
    ],
    out_specs=pl.BlockSpec((G, 2), lambda i: (0, 0)),
    out_shape=jax.ShapeDtypeStruct((G, 2), _F32),
    scratch_shapes=[
        pltpu.VMEM((G, H), _F32),
        pltpu.VMEM((G, 1), _F32),
    ],
)


def kernel(x, edge_index, batch, W1_l, b1, W1_r, W2_l, b2, W2_r, W_out, b_out):
    src = edge_index[0]
    dst = edge_index[1]
    srcm = (jnp.full((EPAD,), N, jnp.int32).at[:E].set(src)
            .reshape(NW, CPW, CHUNK))
    dstm = (jnp.full((EPAD,), N, jnp.int32).at[:E].set(dst)
            .reshape(NW, CPW, CHUNK))
    bat_f = batch.astype(_F32).reshape(N, 1)

    p1, xr = _tc1(x, W1_l, W1_r)
    s1, c1 = _sc_edge_cnt(p1, srcm, dstm)
    p2, h1r = _tc_mid(s1, c1, xr, b1.reshape(1, H), W2_l, W2_r)
    (s2,) = _sc_edge(p2, srcm, dstm)
    return _tc_post(s2, c1, h1r, b2.reshape(1, H), bat_f,
                    W_out, b_out.reshape(1, 2))


# speedup vs baseline: 8.3305x; 1.2702x over previous
import functools

import jax
import jax.numpy as jnp
from jax import lax
from jax.experimental import pallas as pl
from jax.experimental.pallas import tpu as pltpu
from jax.experimental.pallas import tpu_sc as plsc

N = 10000
E = 320000
D = 128
H = 64
G = 128

NW = 32
CHUNK = 128
CPW = 80
EPAD = NW * CPW * CHUNK
NPAD = 10112
ACC = 10240
TPT = ACC // 16

_F32 = jnp.float32


def _mm(a, b):
    return jax.lax.dot_general(a, b, (((1,), (0,)), ((), ())),
                               preferred_element_type=_F32,
                               precision=jax.lax.Precision.HIGHEST)


def _tc1_body(x_ref, wl_ref, wr_ref, p_ref, xr_ref):
    xb = x_ref[...]
    p_ref[...] = _mm(xb, wl_ref[...])
    xr_ref[...] = _mm(xb, wr_ref[...])


_tc1 = pl.pallas_call(
    _tc1_body,
    grid=(8,),
    in_specs=[
        pl.BlockSpec((1264, D), lambda i: (i, 0)),
        pl.BlockSpec((D, H), lambda i: (0, 0)),
        pl.BlockSpec((D, H), lambda i: (0, 0)),
    ],
    out_specs=[
        pl.BlockSpec((1264, H), lambda i: (i, 0)),
        pl.BlockSpec((1264, H), lambda i: (i, 0)),
    ],
    out_shape=[
        jax.ShapeDtypeStruct((NPAD, H), _F32),
        jax.ShapeDtypeStruct((NPAD, H), _F32),
    ],
)


def _sc_body(with_counts, use_tbl, *refs):
    if with_counts:
        (p_hbm, srcm, dstm, out_s, out_c,
         idxs, idxd, rows0, rows1, ones_v, zbuf, zbufc, *rest) = refs
    else:
        (p_hbm, srcm, dstm, out_s,
         idxs, idxd, rows0, rows1, zbuf, *rest) = refs
    if use_tbl:
        tbl = rest[0]
        rest = rest[1:]
    if with_counts:
        (acc, cacc, semg0, semg1, sems0, sems1, semc0, semc1) = rest
    else:
        (acc, semg0, semg1, sems0, sems1) = rest

    cid = lax.axis_index("c")
    sid = lax.axis_index("s")
    wid = sid * 2 + cid
    base = sid * TPT

    zero16 = jnp.zeros((16,), _F32)

    def zfill(i, c):
        for j in range(4):
            zbuf[i, pl.ds(16 * j, 16)] = zero16
        if with_counts:
            zbufc[i, pl.ds(0, 16)] = zero16
            ones_v[i, pl.ds(0, 16)] = jnp.ones((16,), _F32)
            ones_v[i + 64, pl.ds(0, 16)] = jnp.ones((16,), _F32)
        return c

    lax.fori_loop(0, 64, zfill, 0)

    def zcopy(k, c):
        pltpu.sync_copy(zbuf, acc.at[pl.ds(base + k * 64, 64)])
        if with_counts:
            pltpu.sync_copy(zbufc, cacc.at[pl.ds(base + k * 64, 64)])
        return c

    lax.fori_loop(0, TPT // 64, zcopy, 0)

    if use_tbl:
        trows = NPAD // 16
        pltpu.sync_copy(p_hbm.at[pl.ds(sid * trows, trows)],
                        tbl.at[pl.ds(sid * trows, trows)])
        src_tab = tbl
    else:
        src_tab = p_hbm
    plsc.subcore_barrier()

    pltpu.sync_copy(srcm.at[wid], idxs)
    pltpu.sync_copy(dstm.at[wid], idxd)

    def gat(j, rows, sem):
        return pltpu.async_copy(src_tab.at[idxs.at[j]], rows, sem)

    def gat_wait(j, rows, sem):
        pltpu.make_async_copy(src_tab.at[idxs.at[j]], rows, sem).wait()

    def sca(j, rows, sem):
        return pltpu.async_copy(rows, acc.at[idxd.at[j]], sem, add=True)

    def sca_wait(j, rows, sem):
        pltpu.make_async_copy(rows, acc.at[idxd.at[j]], sem).wait()

    def cnt(j, sem):
        return pltpu.async_copy(ones_v, cacc.at[idxd.at[j]], sem, add=True)

    def cnt_wait(j, sem):
        pltpu.make_async_copy(ones_v, cacc.at[idxd.at[j]], sem).wait()

    gat(0, rows0, semg0)
    gat(1, rows1, semg1)

    def edge_pair(k, c):
        j = 2 * k
        gat_wait(j, rows0, semg0)
        sca(j, rows0, sems0)
        if with_counts:
            cnt(j, semc0)
        gat_wait(j + 1, rows1, semg1)
        sca(j + 1, rows1, sems1)
        if with_counts:
            cnt(j + 1, semc1)
        sca_wait(j, rows0, sems0)
        if with_counts:
            cnt_wait(j, semc0)
        gat(j + 2, rows0, semg0)
        sca_wait(j + 1, rows1, sems1)
        if with_counts:
            cnt_wait(j + 1, semc1)
        gat(j + 3, rows1, semg1)
        return c

    lax.fori_loop(0, CPW // 2 - 1, edge_pair, 0)

    j = CPW - 2
    gat_wait(j, rows0, semg0)
    sca(j, rows0, sems0)
    if with_counts:
        cnt(j, semc0)
    gat_wait(j + 1, rows1, semg1)
    sca(j + 1, rows1, sems1)
    if with_counts:
        cnt(j + 1, semc1)
    sca_wait(j, rows0, sems0)
    sca_wait(j + 1, rows1, sems1)
    if with_counts:
        cnt_wait(j, semc0)
        cnt_wait(j + 1, semc1)

    plsc.subcore_barrier()

    pltpu.sync_copy(acc.at[pl.ds(base, TPT)], out_s.at[cid, pl.ds(base, TPT)])
    if with_counts:
        pltpu.sync_copy(cacc.at[pl.ds(base, TPT)],
                        out_c.at[cid, pl.ds(base, TPT)])


def _make_sc(with_counts, use_tbl):
    mesh = plsc.VectorSubcoreMesh(core_axis_name="c", subcore_axis_name="s",
                                  num_cores=2, num_subcores=16)
    out_type = [jax.ShapeDtypeStruct((2, ACC, H), _F32)]
    scratch = [
        pltpu.VMEM((CPW, CHUNK), jnp.int32),
        pltpu.VMEM((CPW, CHUNK), jnp.int32),
        pltpu.VMEM((CHUNK, H), _F32),
        pltpu.VMEM((CHUNK, H), _F32),
    ]
    if with_counts:
        out_type.append(jax.ShapeDtypeStruct((2, ACC, 16), _F32))
        scratch.append(pltpu.VMEM((CHUNK, 16), _F32))
    scratch.append(pltpu.VMEM((64, H), _F32))
    if with_counts:
        scratch.append(pltpu.VMEM((64, 16), _F32))
    if use_tbl:
        scratch.append(pltpu.VMEM_SHARED((NPAD, H), _F32))
    scratch.append(pltpu.VMEM_SHARED((ACC, H), _F32))
    if with_counts:
        scratch.append(pltpu.VMEM_SHARED((ACC, 16), _F32))
    nsem = 6 if with_counts else 4
    scratch.extend([pltpu.SemaphoreType.DMA] * nsem)
    return pl.kernel(
        functools.partial(_sc_body, with_counts, use_tbl),
        out_type=out_type,
        mesh=mesh,
        scratch_types=scratch,
        compiler_params=pltpu.CompilerParams(use_tc_tiling_on_sc=False),
    )


_sc_edge_cnt = _make_sc(True, False)
_sc_edge = _make_sc(False, True)


def _mid_body(s_ref, c_ref, xr_ref, b1_ref, wl_ref, wr_ref, p2_ref, h1r_ref):
    s = s_ref[0] + s_ref[1]
    cnt = c_ref[0, :, 0:1] + c_ref[1, :, 0:1]
    h1 = jnp.maximum(s / jnp.maximum(cnt, 1.0) + xr_ref[...] + b1_ref[...],
                     0.0)
    p2_ref[...] = _mm(h1, wl_ref[...])
    h1r_ref[...] = _mm(h1, wr_ref[...])


_tc_mid = pl.pallas_call(
    _mid_body,
    grid=(8,),
    in_specs=[
        pl.BlockSpec((2, 1264, H), lambda i: (0, i, 0)),
        pl.BlockSpec((2, 1264, 16), lambda i: (0, i, 0)),
        pl.BlockSpec((1264, H), lambda i: (i, 0)),
        pl.BlockSpec((1, H), lambda i: (0, 0)),
        pl.BlockSpec((H, H), lambda i: (0, 0)),
        pl.BlockSpec((H, H), lambda i: (0, 0)),
    ],
    out_specs=[
        pl.BlockSpec((1264, H), lambda i: (i, 0)),
        pl.BlockSpec((1264, H), lambda i: (i, 0)),
    ],
    out_shape=[
        jax.ShapeDtypeStruct((NPAD, H), _F32),
        jax.ShapeDtypeStruct((NPAD, H), _F32),
    ],
)


def _post_body(s_ref, c_ref, h1r_ref, b2_ref, bat_ref, wo_ref, bo_ref,
               out_ref, psum, pcnt):
    i = pl.program_id(0)
    s = s_ref[0] + s_ref[1]
    cnt = c_ref[0, :, 0:1] + c_ref[1, :, 0:1]
    h2 = jnp.maximum(s / jnp.maximum(cnt, 1.0) + h1r_ref[...] + b2_ref[...],
                     0.0)
    bcol = bat_ref[...]
    gids = jax.lax.broadcasted_iota(jnp.int32, (1, G), 1).astype(_F32)
    onehot = (bcol == gids).astype(_F32)
    ps = jax.lax.dot_general(onehot, h2, (((0,), (0,)), ((), ())),
                             preferred_element_type=_F32,
                             precision=jax.lax.Precision.HIGHEST)
    ones_col = jnp.ones_like(bcol)
    pc = jax.lax.dot_general(onehot, ones_col, (((0,), (0,)), ((), ())),
                             preferred_element_type=_F32,
                             precision=jax.lax.Precision.HIGHEST)

    @pl.when(i == 0)
    def _():
        psum[...] = ps
        pcnt[...] = pc

    @pl.when(i > 0)
    def _():
        psum[...] += ps
        pcnt[...] += pc

    @pl.when(i == 9)
    def _():
        pooled = psum[...] / jnp.maximum(pcnt[...], 1.0)
        out_ref[...] = _mm(pooled, wo_ref[...]) + bo_ref[...]


_tc_post = pl.pallas_call(
    _post_body,
    grid=(10,),
    in_specs=[
        pl.BlockSpec((2, 1000, H), lambda i: (0, i, 0)),
        pl.BlockSpec((2, 1000, 16), lambda i: (0, i, 0)),
        pl.BlockSpec((1000, H), lambda i: (i, 0)),
        pl.BlockSpec((1, H), lambda i: (0, 0)),
        pl.BlockSpec((1000, 1), lambda i: (i, 0)),
        pl.BlockSpec((H, 2), lambda i: (0, 0)),
        pl.BlockSpec((1, 2), lambda i: (0, 0)),
    ],
    out_specs=pl.BlockSpec((G, 2), lambda i: (0, 0)),
    out_shape=jax.ShapeDtypeStruct((G, 2), _F32),
    scratch_shapes=[
        pltpu.VMEM((G, H), _F32),
        pltpu.VMEM((G, 1), _F32),
    ],
)


def kernel(x, edge_index, batch, W1_l, b1, W1_r, W2_l, b2, W2_r, W_out, b_out):
    src = edge_index[0]
    dst = edge_index[1]
    srcm = (jnp.full((EPAD,), N, jnp.int32).at[:E].set(src)
            .reshape(NW, CPW, CHUNK))
    dstm = (jnp.full((EPAD,), N, jnp.int32).at[:E].set(dst)
            .reshape(NW, CPW, CHUNK))
    bat_f = batch.astype(_F32).reshape(N, 1)

    p1, xr = _tc1(x, W1_l, W1_r)
    s1, c1 = _sc_edge_cnt(p1, srcm, dstm)
    p2, h1r = _tc_mid(s1, c1, xr, b1.reshape(1, H), W2_l, W2_r)
    (s2,) = _sc_edge(p2, srcm, dstm)
    return _tc_post(s2, c1, h1r, b2.reshape(1, H), bat_f,
                    W_out, b_out.reshape(1, 2))


# speedup vs baseline: 12.4671x; 1.4965x over previous
import jax
import jax.numpy as jnp
from jax import lax
from jax.experimental import pallas as pl
from jax.experimental.pallas import tpu as pltpu
from jax.experimental.pallas import tpu_sc as plsc

N = 10000
E = 320000
D = 128
H = 64
G = 128

NW = 32
CHUNK = 128
CPW = 80
EPAD = NW * CPW * CHUNK
NPAD = 10112
ACC = 10240
TPT = ACC // 16

_F32 = jnp.float32

_SC_MESH = plsc.VectorSubcoreMesh(core_axis_name="c", subcore_axis_name="s",
                                  num_cores=2, num_subcores=16)
_SC_PARAMS = pltpu.CompilerParams(use_tc_tiling_on_sc=False)


def _mm(a, b):
    return jax.lax.dot_general(a, b, (((1,), (0,)), ((), ())),
                               preferred_element_type=_F32,
                               precision=jax.lax.Precision.HIGHEST)


def _tc1_body(x_ref, wl_ref, wr_ref, p_ref, xr_ref):
    xb = x_ref[...]
    p_ref[...] = _mm(xb, wl_ref[...])
    xr_ref[...] = _mm(xb, wr_ref[...])


_tc1 = pl.pallas_call(
    _tc1_body,
    grid=(8,),
    in_specs=[
        pl.BlockSpec((1264, D), lambda i: (i, 0)),
        pl.BlockSpec((D, H), lambda i: (0, 0)),
        pl.BlockSpec((D, H), lambda i: (0, 0)),
    ],
    out_specs=[
        pl.BlockSpec((1264, H), lambda i: (i, 0)),
        pl.BlockSpec((1264, H), lambda i: (i, 0)),
    ],
    out_shape=[
        jax.ShapeDtypeStruct((NPAD, H), _F32),
        jax.ShapeDtypeStruct((NPAD, H), _F32),
    ],
)


def _sc_edge_body(p_hbm, srcm, dstm, out_s,
                  idxs, idxd, rows0, rows1, zbuf, tbl, acc,
                  semg0, semg1, sems0, sems1):
    cid = lax.axis_index("c")
    sid = lax.axis_index("s")
    wid = sid * 2 + cid
    base = sid * TPT

    zero16 = jnp.zeros((16,), _F32)

    def zfill(i, c):
        for j in range(4):
            zbuf[i, pl.ds(16 * j, 16)] = zero16
        return c

    lax.fori_loop(0, 64, zfill, 0)

    def zcopy(k, c):
        pltpu.sync_copy(zbuf, acc.at[pl.ds(base + k * 64, 64)])
        return c

    lax.fori_loop(0, TPT // 64, zcopy, 0)

    trows = NPAD // 16
    pltpu.sync_copy(p_hbm.at[pl.ds(sid * trows, trows)],
                    tbl.at[pl.ds(sid * trows, trows)])
    plsc.subcore_barrier()

    pltpu.sync_copy(srcm.at[wid], idxs)
    pltpu.sync_copy(dstm.at[wid], idxd)

    def gat(j, rows, sem):
        return pltpu.async_copy(tbl.at[idxs.at[j]], rows, sem)

    def gat_wait(j, rows, sem):
        pltpu.make_async_copy(tbl.at[idxs.at[j]], rows, sem).wait()

    def sca(j, rows, sem):
        return pltpu.async_copy(rows, acc.at[idxd.at[j]], sem, add=True)

    def sca_wait(j, rows, sem):
        pltpu.make_async_copy(rows, acc.at[idxd.at[j]], sem).wait()

    gat(0, rows0, semg0)
    gat(1, rows1, semg1)

    def edge_pair(k, c):
        j = 2 * k
        gat_wait(j, rows0, semg0)
        sca(j, rows0, sems0)
        gat_wait(j + 1, rows1, semg1)
        sca(j + 1, rows1, sems1)
        sca_wait(j, rows0, sems0)
        gat(j + 2, rows0, semg0)
        sca_wait(j + 1, rows1, sems1)
        gat(j + 3, rows1, semg1)
        return c

    lax.fori_loop(0, CPW // 2 - 1, edge_pair, 0)

    j = CPW - 2
    gat_wait(j, rows0, semg0)
    sca(j, rows0, sems0)
    gat_wait(j + 1, rows1, semg1)
    sca(j + 1, rows1, sems1)
    sca_wait(j, rows0, sems0)
    sca_wait(j + 1, rows1, sems1)

    plsc.subcore_barrier()
    pltpu.sync_copy(acc.at[pl.ds(base, TPT)], out_s.at[cid, pl.ds(base, TPT)])


_sc_edge = pl.kernel(
    _sc_edge_body,
    out_type=[jax.ShapeDtypeStruct((2, ACC, H), _F32)],
    mesh=_SC_MESH,
    scratch_types=[
        pltpu.VMEM((CPW, CHUNK), jnp.int32),
        pltpu.VMEM((CPW, CHUNK), jnp.int32),
        pltpu.VMEM((CHUNK, H), _F32),
        pltpu.VMEM((CHUNK, H), _F32),
        pltpu.VMEM((64, H), _F32),
        pltpu.VMEM_SHARED((NPAD, H), _F32),
        pltpu.VMEM_SHARED((ACC, H), _F32),
        pltpu.SemaphoreType.DMA,
        pltpu.SemaphoreType.DMA,
        pltpu.SemaphoreType.DMA,
        pltpu.SemaphoreType.DMA,
    ],
    compiler_params=_SC_PARAMS,
)


def _sc_count_body(dstm, out_c, idxd, ones_v, zbufc, cacc, semc0, semc1):
    cid = lax.axis_index("c")
    sid = lax.axis_index("s")
    wid = sid * 2 + cid
    base = sid * TPT

    zero16 = jnp.zeros((16,), _F32)
    one16 = jnp.ones((16,), _F32)

    def zfill(i, c):
        zbufc[i, pl.ds(0, 16)] = zero16
        ones_v[i, pl.ds(0, 16)] = one16
        ones_v[i + 64, pl.ds(0, 16)] = one16
        return c

    lax.fori_loop(0, 64, zfill, 0)

    def zcopy(k, c):
        pltpu.sync_copy(zbufc, cacc.at[pl.ds(base + k * 64, 64)])
        return c

    lax.fori_loop(0, TPT // 64, zcopy, 0)
    plsc.subcore_barrier()

    pltpu.sync_copy(dstm.at[wid], idxd)

    def cnt(j, sem):
        return pltpu.async_copy(ones_v, cacc.at[idxd.at[j]], sem, add=True)

    def cnt_wait(j, sem):
        pltpu.make_async_copy(ones_v, cacc.at[idxd.at[j]], sem).wait()

    cnt(0, semc0)
    cnt(1, semc1)

    def pair(k, c):
        j = 2 * k
        cnt_wait(j, semc0)
        cnt(j + 2, semc0)
        cnt_wait(j + 1, semc1)
        cnt(j + 3, semc1)
        return c

    lax.fori_loop(0, CPW // 2 - 1, pair, 0)
    cnt_wait(CPW - 2, semc0)
    cnt_wait(CPW - 1, semc1)

    plsc.subcore_barrier()
    pltpu.sync_copy(cacc.at[pl.ds(base, TPT)],
                    out_c.at[cid, pl.ds(base, TPT)])


_sc_count = pl.kernel(
    _sc_count_body,
    out_type=[jax.ShapeDtypeStruct((2, ACC, 16), _F32)],
    mesh=_SC_MESH,
    scratch_types=[
        pltpu.VMEM((CPW, CHUNK), jnp.int32),
        pltpu.VMEM((CHUNK, 16), _F32),
        pltpu.VMEM((64, 16), _F32),
        pltpu.VMEM_SHARED((ACC, 16), _F32),
        pltpu.SemaphoreType.DMA,
        pltpu.SemaphoreType.DMA,
    ],
    compiler_params=_SC_PARAMS,
)


def _mid_body(s_ref, c_ref, xr_ref, b1_ref, wl_ref, wr_ref, p2_ref, h1r_ref):
    s = s_ref[0] + s_ref[1]
    cnt = c_ref[0, :, 0:1] + c_ref[1, :, 0:1]
    h1 = jnp.maximum(s / jnp.maximum(cnt, 1.0) + xr_ref[...] + b1_ref[...],
                     0.0)
    p2_ref[...] = _mm(h1, wl_ref[...])
    h1r_ref[...] = _mm(h1, wr_ref[...])


_tc_mid = pl.pallas_call(
    _mid_body,
    grid=(8,),
    in_specs=[
        pl.BlockSpec((2, 1264, H), lambda i: (0, i, 0)),
        pl.BlockSpec((2, 1264, 16), lambda i: (0, i, 0)),
        pl.BlockSpec((1264, H), lambda i: (i, 0)),
        pl.BlockSpec((1, H), lambda i: (0, 0)),
        pl.BlockSpec((H, H), lambda i: (0, 0)),
        pl.BlockSpec((H, H), lambda i: (0, 0)),
    ],
    out_specs=[
        pl.BlockSpec((1264, H), lambda i: (i, 0)),
        pl.BlockSpec((1264, H), lambda i: (i, 0)),
    ],
    out_shape=[
        jax.ShapeDtypeStruct((NPAD, H), _F32),
        jax.ShapeDtypeStruct((NPAD, H), _F32),
    ],
)


def _post_body(s_ref, c_ref, h1r_ref, b2_ref, bat_ref, wo_ref, bo_ref,
               out_ref, psum, pcnt):
    i = pl.program_id(0)
    s = s_ref[0] + s_ref[1]
    cnt = c_ref[0, :, 0:1] + c_ref[1, :, 0:1]
    h2 = jnp.maximum(s / jnp.maximum(cnt, 1.0) + h1r_ref[...] + b2_ref[...],
                     0.0)
    bcol = bat_ref[...]
    gids = jax.lax.broadcasted_iota(jnp.int32, (1, G), 1).astype(_F32)
    onehot = (bcol == gids).astype(_F32)
    ps = jax.lax.dot_general(onehot, h2, (((0,), (0,)), ((), ())),
                             preferred_element_type=_F32,
                             precision=jax.lax.Precision.HIGHEST)
    ones_col = jnp.ones_like(bcol)
    pc = jax.lax.dot_general(onehot, ones_col, (((0,), (0,)), ((), ())),
                             preferred_element_type=_F32,
                             precision=jax.lax.Precision.HIGHEST)

    @pl.when(i == 0)
    def _():
        psum[...] = ps
        pcnt[...] = pc

    @pl.when(i > 0)
    def _():
        psum[...] += ps
        pcnt[...] += pc

    @pl.when(i == 9)
    def _():
        pooled = psum[...] / jnp.maximum(pcnt[...], 1.0)
        out_ref[...] = _mm(pooled, wo_ref[...]) + bo_ref[...]


_tc_post = pl.pallas_call(
    _post_body,
    grid=(10,),
    in_specs=[
        pl.BlockSpec((2, 1000, H), lambda i: (0, i, 0)),
        pl.BlockSpec((2, 1000, 16), lambda i: (0, i, 0)),
        pl.BlockSpec((1000, H), lambda i: (i, 0)),
        pl.BlockSpec((1, H), lambda i: (0, 0)),
        pl.BlockSpec((1000, 1), lambda i: (i, 0)),
        pl.BlockSpec((H, 2), lambda i: (0, 0)),
        pl.BlockSpec((1, 2), lambda i: (0, 0)),
    ],
    out_specs=pl.BlockSpec((G, 2), lambda i: (0, 0)),
    out_shape=jax.ShapeDtypeStruct((G, 2), _F32),
    scratch_shapes=[
        pltpu.VMEM((G, H), _F32),
        pltpu.VMEM((G, 1), _F32),
    ],
)


def kernel(x, edge_index, batch, W1_l, b1, W1_r, W2_l, b2, W2_r, W_out, b_out):
    src = edge_index[0]
    dst = edge_index[1]
    srcm = (jnp.full((EPAD,), N, jnp.int32).at[:E].set(src)
            .reshape(NW, CPW, CHUNK))
    dstm = (jnp.full((EPAD,), N, jnp.int32).at[:E].set(dst)
            .reshape(NW, CPW, CHUNK))
    bat_f = batch.astype(_F32).reshape(N, 1)

    (c1,) = _sc_count(dstm)
    p1, xr = _tc1(x, W1_l, W1_r)
    (s1,) = _sc_edge(p1, srcm, dstm)
    p2, h1r = _tc_mid(s1, c1, xr, b1.reshape(1, H), W2_l, W2_r)
    (s2,) = _sc_edge(p2, srcm, dstm)
    return _tc_post(s2, c1, h1r, b2.reshape(1, H), bat_f,
                    W_out, b_out.reshape(1, 2))
